# Initial kernel scaffold; baseline (speedup 1.0000x reference)
import jax
import jax.numpy as jnp
from jax.experimental import pallas as pl


def kernel(edge_index, edge_dx_, edge_attr, vector_a, vector_b, vector_c, senders_v_t_, senders_w_t_, receivers_v_t_, receivers_w_t_, node_latent, senders_pos, receivers_pos, node_type, params):
    raise NotImplementedError("write your pallas kernel here")


# speedup vs baseline: 2.5639x; 2.5639x over previous
import functools

import jax
import jax.numpy as jnp
from jax import lax
from jax.experimental import pallas as pl
from jax.experimental.pallas import tpu as pltpu
from jax.experimental.pallas import tpu_sc as plsc

N = 10000
E = 320000
D = 128

NC = 2
NS = 16
NW = NC * NS
EC = E // NW
GCH = 400
SCH3 = 200
SCH5 = 400
NPT = N // NS

BE = 1000
BN = 1000

f32 = jnp.float32
i32 = jnp.int32


def _k1_body(nl_hbm, tcol_hbm, s_hbm, r_hbm, nlsum_hbm, gf_hbm,
             tcol_v, sidx_v, ridx_v, rows_v, gf_v, sem):
    core = lax.axis_index("c")
    sub = lax.axis_index("s")
    wid = sub * NC + core
    base_w = wid * EC
    pltpu.sync_copy(tcol_hbm, tcol_v)

    def chunk(ci, carry):
        base = pl.multiple_of(base_w + ci * GCH, GCH)
        pltpu.sync_copy(s_hbm.at[pl.ds(base, GCH)], sidx_v)
        pltpu.sync_copy(r_hbm.at[pl.ds(base, GCH)], ridx_v)
        pltpu.async_copy(nl_hbm.at[sidx_v], rows_v, sem).wait()
        pltpu.async_copy(nl_hbm.at[ridx_v], rows_v, sem, add=True).wait()
        pltpu.sync_copy(rows_v, nlsum_hbm.at[pl.ds(base, GCH), :])
        col0 = jnp.zeros((16,), i32)
        col1 = jnp.ones((16,), i32)
        for i in range(GCH // 16):
            rid = lax.iota(i32, 16) + (i * 16)
            si = sidx_v[pl.ds(i * 16, 16)]
            ri = ridx_v[pl.ds(i * 16, 16)]
            gs = plsc.load_gather(tcol_v, [si])
            gr = plsc.load_gather(tcol_v, [ri])
            plsc.store_scatter(gf_v, [rid, col0], gs)
            plsc.store_scatter(gf_v, [rid, col1], gr)
        pltpu.sync_copy(gf_v, gf_hbm.at[pl.ds(base, GCH), :])
        return carry

    lax.fori_loop(0, EC // GCH, chunk, 0)


@functools.cache
def _get_k1():
  return functools.partial(
    pl.kernel,
    out_type=(jax.ShapeDtypeStruct((E, D), f32),
              jax.ShapeDtypeStruct((E, 8), f32)),
    mesh=plsc.VectorSubcoreMesh(core_axis_name="c", subcore_axis_name="s", num_cores=NC, num_subcores=NS),
    scratch_types=[pltpu.VMEM((N,), f32),
                   pltpu.VMEM((GCH,), i32),
                   pltpu.VMEM((GCH,), i32),
                   pltpu.VMEM((GCH, D), f32),
                   pltpu.VMEM((GCH, 8), f32),
                   pltpu.SemaphoreType.DMA],
    compiler_params=pltpu.CompilerParams(needs_layout_passes=False),
)(_k1_body)


def _k3_body(pay_hbm, r_hbm, z_hbm, out_hbm, acc_sh, pay_v, idx_v):
    core = lax.axis_index("c")
    sub = lax.axis_index("s")
    wid = sub * NC + core
    @pl.when(sub < 10)
    def _zero():
        pltpu.sync_copy(z_hbm.at[pl.ds(sub * 1000, 1000), :],
                        acc_sh.at[pl.ds(sub * 1000, 1000), :])
    plsc.subcore_barrier()

    def chunk(ci, carry):
        base = pl.multiple_of(wid * EC + ci * SCH3, SCH3)
        pltpu.sync_copy(pay_hbm.at[pl.ds(base, SCH3), :], pay_v)
        pltpu.sync_copy(r_hbm.at[pl.ds(base, SCH3)], idx_v)
        pltpu.sync_copy(pay_v, acc_sh.at[idx_v], add=True)
        return carry

    lax.fori_loop(0, EC // SCH3, chunk, 0)
    plsc.subcore_barrier()

    @pl.when(sub < 10)
    def _writeout():
        pltpu.sync_copy(acc_sh.at[pl.ds(sub * 1000, 1000), :],
                        out_hbm.at[pl.ds(core * N + sub * 1000, 1000), :])


@functools.cache
def _get_k3():
  return functools.partial(
    pl.kernel,
    out_type=jax.ShapeDtypeStruct((2 * N, 16), f32),
    mesh=plsc.VectorSubcoreMesh(core_axis_name="c", subcore_axis_name="s", num_cores=NC, num_subcores=NS),
    scratch_types=[pltpu.VMEM_SHARED((N, 16), f32),
                   pltpu.VMEM((SCH3, 16), f32),
                   pltpu.VMEM((SCH3,), i32)],
    compiler_params=pltpu.CompilerParams(use_tc_tiling_on_sc=False),
)(_k3_body)


def _k5_body(mfa_hbm, r_hbm, out_hbm, idx_v, rows_v, sem):
    core = lax.axis_index("c")
    sub = lax.axis_index("s")
    wid = sub * NC + core

    def chunk(ci, carry):
        base = pl.multiple_of(wid * EC + ci * SCH5, SCH5)
        pltpu.sync_copy(r_hbm.at[pl.ds(base, SCH5)], idx_v)
        pltpu.async_copy(mfa_hbm.at[idx_v], rows_v, sem).wait()
        pltpu.sync_copy(rows_v, out_hbm.at[pl.ds(base, SCH5), :])
        return carry

    lax.fori_loop(0, EC // SCH5, chunk, 0)


@functools.cache
def _get_k5():
  return functools.partial(
    pl.kernel,
    out_type=jax.ShapeDtypeStruct((E, 8), f32),
    mesh=plsc.VectorSubcoreMesh(core_axis_name="c", subcore_axis_name="s", num_cores=NC, num_subcores=NS),
    scratch_types=[pltpu.VMEM((SCH5,), i32),
                   pltpu.VMEM((SCH5, 8), f32),
                   pltpu.SemaphoreType.DMA],
    compiler_params=pltpu.CompilerParams(use_tc_tiling_on_sc=False),
)(_k5_body)


def _ln(x, g, b):
    mu = jnp.mean(x, axis=-1, keepdims=True)
    var = jnp.mean((x - mu) ** 2, axis=-1, keepdims=True)
    return (x - mu) / jnp.sqrt(var + 1e-5) * g + b


def _cross(u, v):
    return jnp.concatenate([
        u[:, 1:2] * v[:, 2:3] - u[:, 2:3] * v[:, 1:2],
        u[:, 2:3] * v[:, 0:1] - u[:, 0:1] * v[:, 2:3],
        u[:, 0:1] * v[:, 1:2] - u[:, 1:2] * v[:, 0:1]], axis=1)


def _mm(a, b):
    return jnp.dot(a, b, preferred_element_type=f32)


def _k2_body(dx, ea, va, vb, vc, svt, swt, rvt, rwt, sp_, rp_, nls, gf,
             neW1, neb1, neW2, neb2, neg, nebe,
             eeW1, eeb1, eeW2, eeb2, eeg, eebe,
             ieW1, ieb1, ieW2, ieb2, ieg, iebe,
             dW1, db1, dW2, db2,
             eout, payload):
    vaa = va[...]
    vbb = vb[...]
    vcc = vc[...]
    dxx = dx[...]
    eaa = ea[...]

    def dot3(u, v):
        return jnp.sum(u * v, axis=1, keepdims=True)

    sv = svt[...]
    sw = swt[...]
    rv = rvt[...]
    rw = rwt[...]
    sf = jnp.concatenate([dot3(vaa, sv), dot3(vbb, sv), dot3(vcc, sv),
                          dot3(vaa, sw), dot3(vbb, sw), dot3(vcc, sw)], axis=1)
    rf = -jnp.concatenate([dot3(vaa, rv), dot3(vbb, rv), dot3(vcc, rv),
                           dot3(vaa, rw), dot3(vbb, rw), dot3(vcc, rw)], axis=1)
    both = jnp.concatenate([sf, rf], axis=0)
    h = jnp.maximum(_mm(both, neW1[...]) + neb1[...], 0.0)
    o = _ln(_mm(h, neW2[...]) + neb2[...], neg[...], nebe[...])
    spr = o[:BE] + o[BE:]

    ef = jnp.concatenate([jnp.sqrt(dot3(dxx, dxx)), eaa], axis=1)
    he = jnp.maximum(_mm(ef, eeW1[...]) + eeb1[...], 0.0)
    el = _ln(_mm(he, eeW2[...]) + eeb2[...], eeg[...], eebe[...])

    X = jnp.concatenate([spr, nls[...], el], axis=1)
    hi = jnp.maximum(_mm(X, ieW1[...]) + ieb1[...], 0.0)
    il = _ln(_mm(hi, ieW2[...]) + ieb2[...], ieg[...], iebe[...])

    Hd = jnp.maximum(_mm(il, dW1[...]) + db1[...], 0.0)
    C = _mm(Hd, dW2[...]) + db2[...]

    fraw = C[:, 0:1] * vaa + C[:, 1:2] * vbb + C[:, 2:3] * vcc
    araw = C[:, 8:9] * vaa + C[:, 9:10] * vbb + C[:, 10:11] * vcc
    lam = C[:, 16:17]

    gff = gf[...]
    w = ((eaa[:, 0:1] == -1.0) & (gff[:, 1:2] == -1.0)
         & (gff[:, 0:1] != -1.0)).astype(f32)
    lever = sp_[...] - rp_[...]
    t = (1.0 - w) * araw - _cross(lever, fraw * lam)
    L = lever * (w * lam)
    zpad = jnp.zeros((BE, 8), f32)
    eout[...] = jnp.concatenate([fraw, araw, lam, w, zpad], axis=1)
    payload[...] = jnp.concatenate(
        [w, w * fraw, w * araw, (1.0 - w) * fraw, t, L], axis=1)


def _run_k2(dx, ea, va, vb, vc, svt, swt, rvt, rwt, sp_, rp_, nls, gf, wts):
    g = E // BE
    e3 = pl.BlockSpec((BE, 3), lambda i: (i, 0))
    e4 = pl.BlockSpec((BE, 4), lambda i: (i, 0))
    e8 = pl.BlockSpec((BE, 8), lambda i: (i, 0))
    eD = pl.BlockSpec((BE, D), lambda i: (i, 0))

    def wspec(a):
        return pl.BlockSpec(a.shape, lambda i: tuple(0 for _ in a.shape))

    in_specs = [e3, e4, e3, e3, e3, e3, e3, e3, e3, e3, e3, eD, e8]
    in_specs += [wspec(a) for a in wts]
    return pl.pallas_call(
        _k2_body,
        grid=(g,),
        in_specs=in_specs,
        out_specs=[pl.BlockSpec((BE, 16), lambda i: (i, 0))] * 2,
        out_shape=[jax.ShapeDtypeStruct((E, 16), f32)] * 2,
    )(dx, ea, va, vb, vc, svt, swt, rvt, rwt, sp_, rp_, nls, gf, *wts)


def _k4_body(Sa, Sb, nl, mW1, mb1, mW2, mb2, dv, dw, mfa):
    S = Sa[...] + Sb[...]
    cnt = S[:, 0:1]
    denom = jnp.maximum(cnt, 1.0)
    mean_f = S[:, 1:4] / denom
    mean_a = S[:, 4:7] / denom
    nf = S[:, 7:10]
    ntq = S[:, 10:13] + _cross(S[:, 13:16], mean_f)
    h = jnp.maximum(_mm(nl[...], mW1[...]) + mb1[...], 0.0)
    Cn = _mm(h, mW2[...]) + mb2[...]
    dv[...] = Cn[:, 0:1] * nf
    dw[...] = Cn[:, 8:9] * ntq
    mfa[...] = jnp.concatenate([mean_f, mean_a, jnp.zeros((BN, 2), f32)],
                               axis=1)


def _run_k4(S2, nl, wts):
    g = N // BN
    sspec = pl.BlockSpec((BN, 16), lambda j: (j, 0))
    sspec2 = pl.BlockSpec((BN, 16), lambda j: (j + N // BN, 0))

    def wspec(a):
        return pl.BlockSpec(a.shape, lambda j: tuple(0 for _ in a.shape))

    return pl.pallas_call(
        _k4_body,
        grid=(g,),
        in_specs=[sspec, sspec2, pl.BlockSpec((BN, D), lambda j: (j, 0))]
                 + [wspec(a) for a in wts],
        out_specs=[pl.BlockSpec((BN, 3), lambda j: (j, 0)),
                   pl.BlockSpec((BN, 3), lambda j: (j, 0)),
                   pl.BlockSpec((BN, 8), lambda j: (j, 0))],
        out_shape=[jax.ShapeDtypeStruct((N, 3), f32),
                   jax.ShapeDtypeStruct((N, 3), f32),
                   jax.ShapeDtypeStruct((N, 8), f32)],
    )(S2, S2, nl, *wts)


def _k6_body(eo, mfr, sp_, rp_, fij, tau):
    e = eo[...]
    m = mfr[...]
    fraw = e[:, 0:3]
    araw = e[:, 3:6]
    lam = e[:, 6:7]
    w = e[:, 7:8]
    f = fraw - m[:, 0:3] * w
    a = araw - m[:, 3:6] * w
    lever = sp_[...] - rp_[...]
    fij[...] = f
    tau[...] = a - _cross(lever, f * lam)


def _run_k6(eo, mfr, sp_, rp_):
    g = E // BE
    return pl.pallas_call(
        _k6_body,
        grid=(g,),
        in_specs=[pl.BlockSpec((BE, 16), lambda i: (i, 0)),
                  pl.BlockSpec((BE, 8), lambda i: (i, 0)),
                  pl.BlockSpec((BE, 3), lambda i: (i, 0)),
                  pl.BlockSpec((BE, 3), lambda i: (i, 0))],
        out_specs=[pl.BlockSpec((BE, 3), lambda i: (i, 0)),
                   pl.BlockSpec((BE, 3), lambda i: (i, 0))],
        out_shape=[jax.ShapeDtypeStruct((E, 3), f32),
                   jax.ShapeDtypeStruct((E, 3), f32)],
    )(eo, mfr, sp_, rp_)


def _row(v):
    return v.reshape(1, -1)


def _assemble_weights(params):
    ne = params["node_enc"]
    ee = params["edge_enc"]
    ie = params["inter_enc"]
    i1 = params["i1"]
    i2 = params["i2"]
    fs = params["fs"]
    k2w = [ne["W1"], _row(ne["b1"]), ne["W2"], _row(ne["b2"]),
           _row(ne["g"]), _row(ne["beta"]),
           ee["W1"], _row(ee["b1"]), ee["W2"], _row(ee["b2"]),
           _row(ee["g"]), _row(ee["beta"]),
           ie["W1"], _row(ie["b1"]), ie["W2"], _row(ie["b2"]),
           _row(ie["g"]), _row(ie["beta"])]
    dW1 = jnp.concatenate([i1["W1"], i2["W1"], fs["W1"]], axis=1)
    db1 = _row(jnp.concatenate([i1["b1"], i2["b1"], fs["b1"]]))
    dW2 = jnp.zeros((3 * D, 24), f32)
    dW2 = dW2.at[0:D, 0:3].set(i1["W2"])
    dW2 = dW2.at[D:2 * D, 8:11].set(i2["W2"])
    dW2 = dW2.at[2 * D:3 * D, 16:17].set(fs["W2"])
    db2 = jnp.zeros((24,), f32)
    db2 = db2.at[0:3].set(i1["b2"])
    db2 = db2.at[8:11].set(i2["b2"])
    db2 = db2.at[16:17].set(fs["b2"])
    k2w += [dW1, db1, dW2, _row(db2)]

    im = params["inv_mass"]
    ii = params["inv_inertia"]
    mW1 = jnp.concatenate([im["W1"], ii["W1"]], axis=1)
    mb1 = _row(jnp.concatenate([im["b1"], ii["b1"]]))
    mW2 = jnp.zeros((2 * D, 16), f32)
    mW2 = mW2.at[0:D, 0:1].set(im["W2"])
    mW2 = mW2.at[D:2 * D, 8:9].set(ii["W2"])
    mb2 = jnp.zeros((16,), f32)
    mb2 = mb2.at[0:1].set(im["b2"])
    mb2 = mb2.at[8:9].set(ii["b2"])
    k4w = [mW1, mb1, mW2, _row(mb2)]
    return k2w, k4w


def kernel(edge_index, edge_dx_, edge_attr, vector_a, vector_b, vector_c,
           senders_v_t_, senders_w_t_, receivers_v_t_, receivers_w_t_,
           node_latent, senders_pos, receivers_pos, node_type, params):
    senders = edge_index[0]
    receivers = edge_index[1]
    tcol = node_type[:, -1]
    k2w, k4w = _assemble_weights(params)

    nlsum, gf = _get_k1()(node_latent, tcol, senders, receivers)
    eout, payload = _run_k2(edge_dx_, edge_attr, vector_a, vector_b, vector_c,
                            senders_v_t_, senders_w_t_, receivers_v_t_,
                            receivers_w_t_, senders_pos, receivers_pos,
                            nlsum, gf, k2w)
    S2 = _get_k3()(payload, receivers, jnp.zeros((N, 16), f32))
    dv, dw, mfa = _run_k4(S2, node_latent, k4w)
    mfr = _get_k5()(mfa, receivers)
    fij, tauij = _run_k6(eout, mfr, senders_pos, receivers_pos)
    return (fij, tauij, dv, dw)


# speedup vs baseline: 6.3539x; 2.4782x over previous
import functools

import jax
import jax.numpy as jnp
from jax import lax
from jax.experimental import pallas as pl
from jax.experimental.pallas import tpu as pltpu
from jax.experimental.pallas import tpu_sc as plsc

N = 10000
E = 320000
D = 128

NC = 2
NS = 16
NW = NC * NS
EC = E // NW
GCH = 400
SCH3 = 2000
CH5 = 400

BE = 1280
BN = 1000

f32 = jnp.float32
i32 = jnp.int32


def _k1_body(nl_hbm, tcol_hbm, s_hbm, r_hbm, ea0_hbm, nlsum_hbm, w_hbm,
             tcol_v, sidx_v, ridx_v, rows_v, ea_v, w_v, sem):
    core = lax.axis_index("c")
    sub = lax.axis_index("s")
    wid = sub * NC + core
    base_w = wid * EC
    pltpu.sync_copy(tcol_hbm, tcol_v)

    def chunk(ci, carry):
        base = pl.multiple_of(base_w + ci * GCH, GCH)
        pltpu.sync_copy(s_hbm.at[pl.ds(base, GCH)], sidx_v)
        pltpu.sync_copy(r_hbm.at[pl.ds(base, GCH)], ridx_v)
        pltpu.sync_copy(ea0_hbm.at[pl.ds(base, GCH)], ea_v)
        pltpu.async_copy(nl_hbm.at[sidx_v], rows_v, sem).wait()
        pltpu.async_copy(nl_hbm.at[ridx_v], rows_v, sem, add=True).wait()
        pltpu.sync_copy(rows_v, nlsum_hbm.at[pl.ds(base, GCH), :])
        one = jnp.ones((16,), f32)
        zero = jnp.zeros((16,), f32)
        for i in range(GCH // 16):
            sl = pl.ds(i * 16, 16)
            gs = plsc.load_gather(tcol_v, [sidx_v[sl]])
            gr = plsc.load_gather(tcol_v, [ridx_v[sl]])
            hit = ((ea_v[sl] == -1.0) & (gr == -1.0)) & (gs != -1.0)
            w_v[sl] = jnp.where(hit, one, zero)
        pltpu.sync_copy(w_v, w_hbm.at[pl.ds(base, GCH)])
        return carry

    lax.fori_loop(0, EC // GCH, chunk, 0)


@functools.cache
def _get_k1():
  return functools.partial(
    pl.kernel,
    out_type=(jax.ShapeDtypeStruct((E, D), f32),
              jax.ShapeDtypeStruct((E,), f32)),
    mesh=plsc.VectorSubcoreMesh(core_axis_name="c", subcore_axis_name="s",
                                num_cores=NC, num_subcores=NS),
    scratch_types=[pltpu.VMEM((N,), f32),
                   pltpu.VMEM((GCH,), i32),
                   pltpu.VMEM((GCH,), i32),
                   pltpu.VMEM((GCH, D), f32),
                   pltpu.VMEM((GCH,), f32),
                   pltpu.VMEM((GCH,), f32),
                   pltpu.SemaphoreType.DMA],
    compiler_params=pltpu.CompilerParams(needs_layout_passes=False),
)(_k1_body)


def _k3_body(pay_hbm, r_hbm, z_hbm, out_hbm, acc_sh, pay_v, idx_v):
    core = lax.axis_index("c")
    sub = lax.axis_index("s")
    wid = sub * NC + core

    @pl.when(sub < 10)
    def _zero():
        pltpu.sync_copy(z_hbm.at[pl.ds(sub * 1000, 1000), :],
                        acc_sh.at[pl.ds(sub * 1000, 1000), :])
    plsc.subcore_barrier()

    def chunk(ci, carry):
        base = pl.multiple_of(wid * EC + ci * SCH3, SCH3)
        pltpu.sync_copy(pay_hbm.at[pl.ds(base, SCH3), :], pay_v)
        pltpu.sync_copy(r_hbm.at[pl.ds(base, SCH3)], idx_v)
        pltpu.sync_copy(pay_v, acc_sh.at[idx_v], add=True)
        return carry

    lax.fori_loop(0, EC // SCH3, chunk, 0)
    plsc.subcore_barrier()

    @pl.when(sub < 10)
    def _writeout():
        pltpu.sync_copy(acc_sh.at[pl.ds(sub * 1000, 1000), :],
                        out_hbm.at[pl.ds(core * N + sub * 1000, 1000), :])


@functools.cache
def _get_k3():
  return functools.partial(
    pl.kernel,
    out_type=jax.ShapeDtypeStruct((2 * N, 16), f32),
    mesh=plsc.VectorSubcoreMesh(core_axis_name="c", subcore_axis_name="s",
                                num_cores=NC, num_subcores=NS),
    scratch_types=[pltpu.VMEM_SHARED((N, 16), f32),
                   pltpu.VMEM((SCH3, 16), f32),
                   pltpu.VMEM((SCH3,), i32)],
    compiler_params=pltpu.CompilerParams(use_tc_tiling_on_sc=False),
)(_k3_body)


def _k5_body(mfa_hbm, r_hbm, eoT_hbm, fij_hbm, tau_hbm,
             mfa_v, idx_v, eo_v, f_v, t_v):
    core = lax.axis_index("c")
    sub = lax.axis_index("s")
    wid = sub * NC + core
    pltpu.sync_copy(mfa_hbm, mfa_v)
    c0 = jnp.zeros((16,), i32)

    def chunk(ci, carry):
        base = pl.multiple_of(wid * EC + ci * CH5, CH5)
        pltpu.sync_copy(r_hbm.at[pl.ds(base, CH5)], idx_v)
        pltpu.sync_copy(eoT_hbm.at[:, pl.ds(base, CH5)], eo_v)
        for k in range(CH5 // 16):
            sl = pl.ds(k * 16, 16)
            rid = lax.iota(i32, 16) + (k * 16)
            ridx = idx_v[sl]
            e0 = eo_v[0, sl]
            e1 = eo_v[1, sl]
            e2 = eo_v[2, sl]
            a0 = eo_v[3, sl]
            a1 = eo_v[4, sl]
            a2 = eo_v[5, sl]
            lam = eo_v[6, sl]
            w = eo_v[7, sl]
            l0 = eo_v[8, sl]
            l1 = eo_v[9, sl]
            l2 = eo_v[10, sl]
            mf0 = plsc.load_gather(mfa_v, [ridx, c0])
            mf1 = plsc.load_gather(mfa_v, [ridx, c0 + 1])
            mf2 = plsc.load_gather(mfa_v, [ridx, c0 + 2])
            ma0 = plsc.load_gather(mfa_v, [ridx, c0 + 3])
            ma1 = plsc.load_gather(mfa_v, [ridx, c0 + 4])
            ma2 = plsc.load_gather(mfa_v, [ridx, c0 + 5])
            f0 = e0 - mf0 * w
            f1 = e1 - mf1 * w
            f2 = e2 - mf2 * w
            g0 = f0 * lam
            g1 = f1 * lam
            g2 = f2 * lam
            t0 = (a0 - ma0 * w) - (l1 * g2 - l2 * g1)
            t1 = (a1 - ma1 * w) - (l2 * g0 - l0 * g2)
            t2 = (a2 - ma2 * w) - (l0 * g1 - l1 * g0)
            plsc.store_scatter(f_v, [rid, c0], f0)
            plsc.store_scatter(f_v, [rid, c0 + 1], f1)
            plsc.store_scatter(f_v, [rid, c0 + 2], f2)
            plsc.store_scatter(t_v, [rid, c0], t0)
            plsc.store_scatter(t_v, [rid, c0 + 1], t1)
            plsc.store_scatter(t_v, [rid, c0 + 2], t2)
        pltpu.sync_copy(f_v, fij_hbm.at[pl.ds(base, CH5), :])
        pltpu.sync_copy(t_v, tau_hbm.at[pl.ds(base, CH5), :])
        return carry

    lax.fori_loop(0, EC // CH5, chunk, 0)


@functools.cache
def _get_k5():
  return functools.partial(
    pl.kernel,
    out_type=(jax.ShapeDtypeStruct((E, 3), f32),
              jax.ShapeDtypeStruct((E, 3), f32)),
    mesh=plsc.VectorSubcoreMesh(core_axis_name="c", subcore_axis_name="s",
                                num_cores=NC, num_subcores=NS),
    scratch_types=[pltpu.VMEM((N, 8), f32),
                   pltpu.VMEM((CH5,), i32),
                   pltpu.VMEM((16, CH5), f32),
                   pltpu.VMEM((CH5, 3), f32),
                   pltpu.VMEM((CH5, 3), f32)],
    compiler_params=pltpu.CompilerParams(use_tc_tiling_on_sc=False,
                                         needs_layout_passes=False),
)(_k5_body)


def _ln(x, g, b):
    mu = jnp.mean(x, axis=-1, keepdims=True)
    var = jnp.mean((x - mu) ** 2, axis=-1, keepdims=True)
    return (x - mu) / jnp.sqrt(var + 1e-5) * g + b


def _cross(u, v):
    return jnp.concatenate([
        u[:, 1:2] * v[:, 2:3] - u[:, 2:3] * v[:, 1:2],
        u[:, 2:3] * v[:, 0:1] - u[:, 0:1] * v[:, 2:3],
        u[:, 0:1] * v[:, 1:2] - u[:, 1:2] * v[:, 0:1]], axis=1)


def _mm(a, b):
    return jnp.dot(a, b, preferred_element_type=f32)


def _k2_body(vt, w3, nls,
             neW1, neb1, neW2, neb2, neg, nebe,
             eeW1, eeb1, eeW2, eeb2, eeg, eebe,
             ieW1, ieb1, ieW2, ieb2, ieg, iebe,
             dW1, db1, dW2, db2,
             payload, eoT):
    V = vt[...]

    def row(j):
        return V[j:j + 1, :]

    def tdot(i, j):
        return (row(i) * row(j) + row(i + 1) * row(j + 1)
                + row(i + 2) * row(j + 2))

    sfT = jnp.concatenate([tdot(0, 9), tdot(3, 9), tdot(6, 9),
                           tdot(0, 12), tdot(3, 12), tdot(6, 12)], axis=0)
    rfT = jnp.concatenate([tdot(0, 15), tdot(3, 15), tdot(6, 15),
                           tdot(0, 18), tdot(3, 18), tdot(6, 18)], axis=0)
    both = jnp.concatenate([sfT.T, -rfT.T], axis=0)
    h = jnp.maximum(_mm(both, neW1[...]) + neb1[...], 0.0)
    o = _ln(_mm(h, neW2[...]) + neb2[...], neg[...], nebe[...])
    spr = o[:BE] + o[BE:]

    nrm = jnp.sqrt(tdot(21, 21))
    efT = jnp.concatenate([nrm, V[30:34, :]], axis=0)
    he = jnp.maximum(_mm(efT.T, eeW1[...]) + eeb1[...], 0.0)
    el = _ln(_mm(he, eeW2[...]) + eeb2[...], eeg[...], eebe[...])

    X = jnp.concatenate([spr, nls[...], el], axis=1)
    hi = jnp.maximum(_mm(X, ieW1[...]) + ieb1[...], 0.0)
    il = _ln(_mm(hi, ieW2[...]) + ieb2[...], ieg[...], iebe[...])

    Hd = jnp.maximum(_mm(il, dW1[...]) + db1[...], 0.0)
    C = (_mm(Hd, dW2[...]) + db2[...]).T

    def crow(j):
        return C[j:j + 1, :]

    f = [crow(0) * row(c) + crow(1) * row(3 + c) + crow(2) * row(6 + c)
         for c in range(3)]
    a = [crow(8) * row(c) + crow(9) * row(3 + c) + crow(10) * row(6 + c)
         for c in range(3)]
    lam = crow(16)
    w = w3[...][0]
    lv = [row(24 + c) - row(27 + c) for c in range(3)]
    g = [f[c] * lam for c in range(3)]
    mom = [lv[1] * g[2] - lv[2] * g[1],
           lv[2] * g[0] - lv[0] * g[2],
           lv[0] * g[1] - lv[1] * g[0]]
    omw = 1.0 - w
    wl = w * lam
    payT = jnp.concatenate(
        [w, w * f[0], w * f[1], w * f[2], w * a[0], w * a[1], w * a[2],
         omw * f[0], omw * f[1], omw * f[2],
         omw * a[0] - mom[0], omw * a[1] - mom[1], omw * a[2] - mom[2],
         lv[0] * wl, lv[1] * wl, lv[2] * wl], axis=0)
    payload[...] = payT.T
    eoT[...] = jnp.concatenate(
        [f[0], f[1], f[2], a[0], a[1], a[2], lam, w,
         lv[0], lv[1], lv[2], jnp.zeros((5, BE), f32)], axis=0)


def _run_k2(vt, w3, nls, wts):
    g = E // BE

    def wspec(a):
        return pl.BlockSpec(a.shape, lambda i: tuple(0 for _ in a.shape))

    in_specs = [pl.BlockSpec((34, BE), lambda i: (0, i)),
                pl.BlockSpec((1, 1, BE), lambda i: (i, 0, 0)),
                pl.BlockSpec((BE, D), lambda i: (i, 0))]
    in_specs += [wspec(a) for a in wts]
    return pl.pallas_call(
        _k2_body,
        grid=(g,),
        in_specs=in_specs,
        out_specs=[pl.BlockSpec((BE, 16), lambda i: (i, 0)),
                   pl.BlockSpec((16, BE), lambda i: (0, i))],
        out_shape=[jax.ShapeDtypeStruct((E, 16), f32),
                   jax.ShapeDtypeStruct((16, E), f32)],
    )(vt, w3, nls, *wts)


def _k4_body(Sa, Sb, nl, mW1, mb1, mW2, mb2, dv, dw, mfa):
    S = Sa[...] + Sb[...]
    cnt = S[:, 0:1]
    denom = jnp.maximum(cnt, 1.0)
    mean_f = S[:, 1:4] / denom
    mean_a = S[:, 4:7] / denom
    nf = S[:, 7:10]
    ntq = S[:, 10:13] + _cross(S[:, 13:16], mean_f)
    h = jnp.maximum(_mm(nl[...], mW1[...]) + mb1[...], 0.0)
    Cn = _mm(h, mW2[...]) + mb2[...]
    dv[...] = Cn[:, 0:1] * nf
    dw[...] = Cn[:, 8:9] * ntq
    mfa[...] = jnp.concatenate([mean_f, mean_a, jnp.zeros((BN, 2), f32)],
                               axis=1)


def _run_k4(S2, nl, wts):
    g = N // BN
    sspec = pl.BlockSpec((BN, 16), lambda j: (j, 0))
    sspec2 = pl.BlockSpec((BN, 16), lambda j: (j + N // BN, 0))

    def wspec(a):
        return pl.BlockSpec(a.shape, lambda j: tuple(0 for _ in a.shape))

    return pl.pallas_call(
        _k4_body,
        grid=(g,),
        in_specs=[sspec, sspec2, pl.BlockSpec((BN, D), lambda j: (j, 0))]
                 + [wspec(a) for a in wts],
        out_specs=[pl.BlockSpec((BN, 3), lambda j: (j, 0)),
                   pl.BlockSpec((BN, 3), lambda j: (j, 0)),
                   pl.BlockSpec((BN, 8), lambda j: (j, 0))],
        out_shape=[jax.ShapeDtypeStruct((N, 3), f32),
                   jax.ShapeDtypeStruct((N, 3), f32),
                   jax.ShapeDtypeStruct((N, 8), f32)],
    )(S2, S2, nl, *wts)


def _row(v):
    return v.reshape(1, -1)


def _assemble_weights(params):
    ne = params["node_enc"]
    ee = params["edge_enc"]
    ie = params["inter_enc"]
    i1 = params["i1"]
    i2 = params["i2"]
    fs = params["fs"]
    k2w = [ne["W1"], _row(ne["b1"]), ne["W2"], _row(ne["b2"]),
           _row(ne["g"]), _row(ne["beta"]),
           ee["W1"], _row(ee["b1"]), ee["W2"], _row(ee["b2"]),
           _row(ee["g"]), _row(ee["beta"]),
           ie["W1"], _row(ie["b1"]), ie["W2"], _row(ie["b2"]),
           _row(ie["g"]), _row(ie["beta"])]
    dW1 = jnp.concatenate([i1["W1"], i2["W1"], fs["W1"]], axis=1)
    db1 = _row(jnp.concatenate([i1["b1"], i2["b1"], fs["b1"]]))
    dW2 = jnp.zeros((3 * D, 24), f32)
    dW2 = dW2.at[0:D, 0:3].set(i1["W2"])
    dW2 = dW2.at[D:2 * D, 8:11].set(i2["W2"])
    dW2 = dW2.at[2 * D:3 * D, 16:17].set(fs["W2"])
    db2 = jnp.zeros((24,), f32)
    db2 = db2.at[0:3].set(i1["b2"])
    db2 = db2.at[8:11].set(i2["b2"])
    db2 = db2.at[16:17].set(fs["b2"])
    k2w += [dW1, db1, dW2, _row(db2)]

    im = params["inv_mass"]
    ii = params["inv_inertia"]
    mW1 = jnp.concatenate([im["W1"], ii["W1"]], axis=1)
    mb1 = _row(jnp.concatenate([im["b1"], ii["b1"]]))
    mW2 = jnp.zeros((2 * D, 16), f32)
    mW2 = mW2.at[0:D, 0:1].set(im["W2"])
    mW2 = mW2.at[D:2 * D, 8:9].set(ii["W2"])
    mb2 = jnp.zeros((16,), f32)
    mb2 = mb2.at[0:1].set(im["b2"])
    mb2 = mb2.at[8:9].set(ii["b2"])
    k4w = [mW1, mb1, mW2, _row(mb2)]
    return k2w, k4w


def kernel(edge_index, edge_dx_, edge_attr, vector_a, vector_b, vector_c,
           senders_v_t_, senders_w_t_, receivers_v_t_, receivers_w_t_,
           node_latent, senders_pos, receivers_pos, node_type, params):
    senders = edge_index[0]
    receivers = edge_index[1]
    tcol = node_type[:, -1]
    ea0 = edge_attr[:, 0]
    k2w, k4w = _assemble_weights(params)
    vt = jnp.concatenate(
        [vector_a.T, vector_b.T, vector_c.T, senders_v_t_.T, senders_w_t_.T,
         receivers_v_t_.T, receivers_w_t_.T, edge_dx_.T, senders_pos.T,
         receivers_pos.T, edge_attr.T], axis=0)

    nlsum, w = _get_k1()(node_latent, tcol, senders, receivers, ea0)
    w3 = w.reshape(E // BE, 1, BE)
    payload, eoT = _run_k2(vt, w3, nlsum, k2w)
    S2 = _get_k3()(payload, receivers, jnp.zeros((N, 16), f32))
    dv, dw, mfa = _run_k4(S2, node_latent, k4w)
    fij, tauij = _get_k5()(mfa, receivers, eoT)
    return (fij, tauij, dv, dw)


# speedup vs baseline: 6.3981x; 1.0069x over previous
import functools

import jax
import jax.numpy as jnp
from jax import lax
from jax.experimental import pallas as pl
from jax.experimental.pallas import tpu as pltpu
from jax.experimental.pallas import tpu_sc as plsc

N = 10000
E = 320000
D = 128

NC = 2
NS = 16
NW = NC * NS
EC = E // NW
GCH = 400
SCH3 = 2000
CH5 = 400

BE = 1280
BN = 1000

f32 = jnp.float32
i32 = jnp.int32


def _k1_body(nl_hbm, tcol_hbm, s_hbm, r_hbm, ea0_hbm, nlsum_hbm, w_hbm,
             tcol_v, sidx_v, ridx_v, rows_v, ea_v, w_v, sem):
    core = lax.axis_index("c")
    sub = lax.axis_index("s")
    wid = sub * NC + core
    base_w = wid * EC
    pltpu.sync_copy(tcol_hbm, tcol_v)

    def chunk(ci, carry):
        base = pl.multiple_of(base_w + ci * GCH, GCH)
        pltpu.sync_copy(s_hbm.at[pl.ds(base, GCH)], sidx_v)
        pltpu.sync_copy(r_hbm.at[pl.ds(base, GCH)], ridx_v)
        pltpu.sync_copy(ea0_hbm.at[pl.ds(base, GCH)], ea_v)
        pltpu.async_copy(nl_hbm.at[sidx_v], rows_v, sem).wait()
        pltpu.async_copy(nl_hbm.at[ridx_v], rows_v, sem, add=True).wait()
        pltpu.sync_copy(rows_v, nlsum_hbm.at[pl.ds(base, GCH), :])
        one = jnp.ones((16,), f32)
        zero = jnp.zeros((16,), f32)
        for i in range(GCH // 16):
            sl = pl.ds(i * 16, 16)
            gs = plsc.load_gather(tcol_v, [sidx_v[sl]])
            gr = plsc.load_gather(tcol_v, [ridx_v[sl]])
            hit = ((ea_v[sl] == -1.0) & (gr == -1.0)) & (gs != -1.0)
            w_v[sl] = jnp.where(hit, one, zero)
        pltpu.sync_copy(w_v, w_hbm.at[pl.ds(base, GCH)])
        return carry

    lax.fori_loop(0, EC // GCH, chunk, 0)


@functools.cache
def _get_k1():
  return functools.partial(
    pl.kernel,
    out_type=(jax.ShapeDtypeStruct((E, D), f32),
              jax.ShapeDtypeStruct((E,), f32)),
    mesh=plsc.VectorSubcoreMesh(core_axis_name="c", subcore_axis_name="s",
                                num_cores=NC, num_subcores=NS),
    scratch_types=[pltpu.VMEM((N,), f32),
                   pltpu.VMEM((GCH,), i32),
                   pltpu.VMEM((GCH,), i32),
                   pltpu.VMEM((GCH, D), f32),
                   pltpu.VMEM((GCH,), f32),
                   pltpu.VMEM((GCH,), f32),
                   pltpu.SemaphoreType.DMA],
    compiler_params=pltpu.CompilerParams(needs_layout_passes=False),
)(_k1_body)


def _k3_body(pay_hbm, r_hbm, z_hbm, out_hbm, acc_sh, pay_v, idx_v):
    core = lax.axis_index("c")
    sub = lax.axis_index("s")
    wid = sub * NC + core

    @pl.when(sub < 10)
    def _zero():
        pltpu.sync_copy(z_hbm.at[pl.ds(sub * 1000, 1000), :],
                        acc_sh.at[pl.ds(sub * 1000, 1000), :])
    plsc.subcore_barrier()

    def chunk(ci, carry):
        base = pl.multiple_of(wid * EC + ci * SCH3, SCH3)
        pltpu.sync_copy(pay_hbm.at[pl.ds(base, SCH3), :], pay_v)
        pltpu.sync_copy(r_hbm.at[pl.ds(base, SCH3)], idx_v)
        pltpu.sync_copy(pay_v, acc_sh.at[idx_v], add=True)
        return carry

    lax.fori_loop(0, EC // SCH3, chunk, 0)
    plsc.subcore_barrier()

    @pl.when(sub < 10)
    def _writeout():
        pltpu.sync_copy(acc_sh.at[pl.ds(sub * 1000, 1000), :],
                        out_hbm.at[pl.ds(core * N + sub * 1000, 1000), :])


@functools.cache
def _get_k3():
  return functools.partial(
    pl.kernel,
    out_type=jax.ShapeDtypeStruct((2 * N, 16), f32),
    mesh=plsc.VectorSubcoreMesh(core_axis_name="c", subcore_axis_name="s",
                                num_cores=NC, num_subcores=NS),
    scratch_types=[pltpu.VMEM_SHARED((N, 16), f32),
                   pltpu.VMEM((SCH3, 16), f32),
                   pltpu.VMEM((SCH3,), i32)],
    compiler_params=pltpu.CompilerParams(use_tc_tiling_on_sc=False),
)(_k3_body)


def _k5_body(mfa_hbm, r_hbm, eoT_hbm, fij_hbm, tau_hbm,
             mfa_v, idx_v, eo_v, f_v, t_v):
    core = lax.axis_index("c")
    sub = lax.axis_index("s")
    wid = sub * NC + core
    pltpu.sync_copy(mfa_hbm, mfa_v)
    c0 = jnp.zeros((16,), i32)

    def chunk(ci, carry):
        base = pl.multiple_of(wid * EC + ci * CH5, CH5)
        pltpu.sync_copy(r_hbm.at[pl.ds(base, CH5)], idx_v)
        pltpu.sync_copy(eoT_hbm.at[:, pl.ds(base, CH5)], eo_v)
        for k in range(CH5 // 16):
            sl = pl.ds(k * 16, 16)
            rid = lax.iota(i32, 16) + (k * 16)
            ridx = idx_v[sl]
            e0 = eo_v[0, sl]
            e1 = eo_v[1, sl]
            e2 = eo_v[2, sl]
            a0 = eo_v[3, sl]
            a1 = eo_v[4, sl]
            a2 = eo_v[5, sl]
            lam = eo_v[6, sl]
            w = eo_v[7, sl]
            l0 = eo_v[8, sl]
            l1 = eo_v[9, sl]
            l2 = eo_v[10, sl]
            mf0 = plsc.load_gather(mfa_v, [ridx, c0])
            mf1 = plsc.load_gather(mfa_v, [ridx, c0 + 1])
            mf2 = plsc.load_gather(mfa_v, [ridx, c0 + 2])
            ma0 = plsc.load_gather(mfa_v, [ridx, c0 + 3])
            ma1 = plsc.load_gather(mfa_v, [ridx, c0 + 4])
            ma2 = plsc.load_gather(mfa_v, [ridx, c0 + 5])
            f0 = e0 - mf0 * w
            f1 = e1 - mf1 * w
            f2 = e2 - mf2 * w
            g0 = f0 * lam
            g1 = f1 * lam
            g2 = f2 * lam
            t0 = (a0 - ma0 * w) - (l1 * g2 - l2 * g1)
            t1 = (a1 - ma1 * w) - (l2 * g0 - l0 * g2)
            t2 = (a2 - ma2 * w) - (l0 * g1 - l1 * g0)
            plsc.store_scatter(f_v, [rid, c0], f0)
            plsc.store_scatter(f_v, [rid, c0 + 1], f1)
            plsc.store_scatter(f_v, [rid, c0 + 2], f2)
            plsc.store_scatter(t_v, [rid, c0], t0)
            plsc.store_scatter(t_v, [rid, c0 + 1], t1)
            plsc.store_scatter(t_v, [rid, c0 + 2], t2)
        pltpu.sync_copy(f_v, fij_hbm.at[pl.ds(base, CH5), :])
        pltpu.sync_copy(t_v, tau_hbm.at[pl.ds(base, CH5), :])
        return carry

    lax.fori_loop(0, EC // CH5, chunk, 0)


@functools.cache
def _get_k5():
  return functools.partial(
    pl.kernel,
    out_type=(jax.ShapeDtypeStruct((E, 3), f32),
              jax.ShapeDtypeStruct((E, 3), f32)),
    mesh=plsc.VectorSubcoreMesh(core_axis_name="c", subcore_axis_name="s",
                                num_cores=NC, num_subcores=NS),
    scratch_types=[pltpu.VMEM((N, 8), f32),
                   pltpu.VMEM((CH5,), i32),
                   pltpu.VMEM((16, CH5), f32),
                   pltpu.VMEM((CH5, 3), f32),
                   pltpu.VMEM((CH5, 3), f32)],
    compiler_params=pltpu.CompilerParams(use_tc_tiling_on_sc=False,
                                         needs_layout_passes=False),
)(_k5_body)


def _ln(x, g, b):
    mu = jnp.mean(x, axis=-1, keepdims=True)
    var = jnp.mean((x - mu) ** 2, axis=-1, keepdims=True)
    return (x - mu) / jnp.sqrt(var + 1e-5) * g + b


def _cross(u, v):
    return jnp.concatenate([
        u[:, 1:2] * v[:, 2:3] - u[:, 2:3] * v[:, 1:2],
        u[:, 2:3] * v[:, 0:1] - u[:, 0:1] * v[:, 2:3],
        u[:, 0:1] * v[:, 1:2] - u[:, 1:2] * v[:, 0:1]], axis=1)


def _mm(a, b):
    return jnp.dot(a.astype(jnp.bfloat16), b.astype(jnp.bfloat16),
                   preferred_element_type=f32)


def _k2_body(vt, w3, nls,
             neW1, neb1, neW2, neb2, neg, nebe,
             eeW1, eeb1, eeW2, eeb2, eeg, eebe,
             ieW1, ieb1, ieW2, ieb2, ieg, iebe,
             dW1, db1, dW2, db2,
             payload, eoT):
    V = vt[...]

    def row(j):
        return V[j:j + 1, :]

    def tdot(i, j):
        return (row(i) * row(j) + row(i + 1) * row(j + 1)
                + row(i + 2) * row(j + 2))

    sfT = jnp.concatenate([tdot(0, 9), tdot(3, 9), tdot(6, 9),
                           tdot(0, 12), tdot(3, 12), tdot(6, 12)], axis=0)
    rfT = jnp.concatenate([tdot(0, 15), tdot(3, 15), tdot(6, 15),
                           tdot(0, 18), tdot(3, 18), tdot(6, 18)], axis=0)
    both = jnp.concatenate([sfT.T, -rfT.T], axis=0)
    h = jnp.maximum(_mm(both, neW1[...]) + neb1[...], 0.0)
    o = _ln(_mm(h, neW2[...]) + neb2[...], neg[...], nebe[...])
    spr = o[:BE] + o[BE:]

    nrm = jnp.sqrt(tdot(21, 21))
    efT = jnp.concatenate([nrm, V[30:34, :]], axis=0)
    he = jnp.maximum(_mm(efT.T, eeW1[...]) + eeb1[...], 0.0)
    el = _ln(_mm(he, eeW2[...]) + eeb2[...], eeg[...], eebe[...])

    X = jnp.concatenate([spr, nls[...], el], axis=1)
    hi = jnp.maximum(_mm(X, ieW1[...]) + ieb1[...], 0.0)
    il = _ln(_mm(hi, ieW2[...]) + ieb2[...], ieg[...], iebe[...])

    Hd = jnp.maximum(_mm(il, dW1[...]) + db1[...], 0.0)
    C = (_mm(Hd, dW2[...]) + db2[...]).T

    def crow(j):
        return C[j:j + 1, :]

    f = [crow(0) * row(c) + crow(1) * row(3 + c) + crow(2) * row(6 + c)
         for c in range(3)]
    a = [crow(8) * row(c) + crow(9) * row(3 + c) + crow(10) * row(6 + c)
         for c in range(3)]
    lam = crow(16)
    w = w3[...][0]
    lv = [row(24 + c) - row(27 + c) for c in range(3)]
    g = [f[c] * lam for c in range(3)]
    mom = [lv[1] * g[2] - lv[2] * g[1],
           lv[2] * g[0] - lv[0] * g[2],
           lv[0] * g[1] - lv[1] * g[0]]
    omw = 1.0 - w
    wl = w * lam
    payT = jnp.concatenate(
        [w, w * f[0], w * f[1], w * f[2], w * a[0], w * a[1], w * a[2],
         omw * f[0], omw * f[1], omw * f[2],
         omw * a[0] - mom[0], omw * a[1] - mom[1], omw * a[2] - mom[2],
         lv[0] * wl, lv[1] * wl, lv[2] * wl], axis=0)
    payload[...] = payT.T
    eoT[...] = jnp.concatenate(
        [f[0], f[1], f[2], a[0], a[1], a[2], lam, w,
         lv[0], lv[1], lv[2], jnp.zeros((5, BE), f32)], axis=0)


def _run_k2(vt, w3, nls, wts):
    g = E // BE

    def wspec(a):
        return pl.BlockSpec(a.shape, lambda i: tuple(0 for _ in a.shape))

    in_specs = [pl.BlockSpec((34, BE), lambda i: (0, i)),
                pl.BlockSpec((1, 1, BE), lambda i: (i, 0, 0)),
                pl.BlockSpec((BE, D), lambda i: (i, 0))]
    in_specs += [wspec(a) for a in wts]
    return pl.pallas_call(
        _k2_body,
        grid=(g,),
        in_specs=in_specs,
        out_specs=[pl.BlockSpec((BE, 16), lambda i: (i, 0)),
                   pl.BlockSpec((16, BE), lambda i: (0, i))],
        out_shape=[jax.ShapeDtypeStruct((E, 16), f32),
                   jax.ShapeDtypeStruct((16, E), f32)],
    )(vt, w3, nls, *wts)


def _k4_body(Sa, Sb, nl, mW1, mb1, mW2, mb2, dv, dw, mfa):
    S = Sa[...] + Sb[...]
    cnt = S[:, 0:1]
    denom = jnp.maximum(cnt, 1.0)
    mean_f = S[:, 1:4] / denom
    mean_a = S[:, 4:7] / denom
    nf = S[:, 7:10]
    ntq = S[:, 10:13] + _cross(S[:, 13:16], mean_f)
    h = jnp.maximum(_mm(nl[...], mW1[...]) + mb1[...], 0.0)
    Cn = _mm(h, mW2[...]) + mb2[...]
    dv[...] = Cn[:, 0:1] * nf
    dw[...] = Cn[:, 8:9] * ntq
    mfa[...] = jnp.concatenate([mean_f, mean_a, jnp.zeros((BN, 2), f32)],
                               axis=1)


def _run_k4(S2, nl, wts):
    g = N // BN
    sspec = pl.BlockSpec((BN, 16), lambda j: (j, 0))
    sspec2 = pl.BlockSpec((BN, 16), lambda j: (j + N // BN, 0))

    def wspec(a):
        return pl.BlockSpec(a.shape, lambda j: tuple(0 for _ in a.shape))

    return pl.pallas_call(
        _k4_body,
        grid=(g,),
        in_specs=[sspec, sspec2, pl.BlockSpec((BN, D), lambda j: (j, 0))]
                 + [wspec(a) for a in wts],
        out_specs=[pl.BlockSpec((BN, 3), lambda j: (j, 0)),
                   pl.BlockSpec((BN, 3), lambda j: (j, 0)),
                   pl.BlockSpec((BN, 8), lambda j: (j, 0))],
        out_shape=[jax.ShapeDtypeStruct((N, 3), f32),
                   jax.ShapeDtypeStruct((N, 3), f32),
                   jax.ShapeDtypeStruct((N, 8), f32)],
    )(S2, S2, nl, *wts)


def _row(v):
    return v.reshape(1, -1)


def _assemble_weights(params):
    ne = params["node_enc"]
    ee = params["edge_enc"]
    ie = params["inter_enc"]
    i1 = params["i1"]
    i2 = params["i2"]
    fs = params["fs"]
    k2w = [ne["W1"], _row(ne["b1"]), ne["W2"], _row(ne["b2"]),
           _row(ne["g"]), _row(ne["beta"]),
           ee["W1"], _row(ee["b1"]), ee["W2"], _row(ee["b2"]),
           _row(ee["g"]), _row(ee["beta"]),
           ie["W1"], _row(ie["b1"]), ie["W2"], _row(ie["b2"]),
           _row(ie["g"]), _row(ie["beta"])]
    dW1 = jnp.concatenate([i1["W1"], i2["W1"], fs["W1"]], axis=1)
    db1 = _row(jnp.concatenate([i1["b1"], i2["b1"], fs["b1"]]))
    dW2 = jnp.zeros((3 * D, 24), f32)
    dW2 = dW2.at[0:D, 0:3].set(i1["W2"])
    dW2 = dW2.at[D:2 * D, 8:11].set(i2["W2"])
    dW2 = dW2.at[2 * D:3 * D, 16:17].set(fs["W2"])
    db2 = jnp.zeros((24,), f32)
    db2 = db2.at[0:3].set(i1["b2"])
    db2 = db2.at[8:11].set(i2["b2"])
    db2 = db2.at[16:17].set(fs["b2"])
    k2w += [dW1, db1, dW2, _row(db2)]

    im = params["inv_mass"]
    ii = params["inv_inertia"]
    mW1 = jnp.concatenate([im["W1"], ii["W1"]], axis=1)
    mb1 = _row(jnp.concatenate([im["b1"], ii["b1"]]))
    mW2 = jnp.zeros((2 * D, 16), f32)
    mW2 = mW2.at[0:D, 0:1].set(im["W2"])
    mW2 = mW2.at[D:2 * D, 8:9].set(ii["W2"])
    mb2 = jnp.zeros((16,), f32)
    mb2 = mb2.at[0:1].set(im["b2"])
    mb2 = mb2.at[8:9].set(ii["b2"])
    k4w = [mW1, mb1, mW2, _row(mb2)]
    return k2w, k4w


def kernel(edge_index, edge_dx_, edge_attr, vector_a, vector_b, vector_c,
           senders_v_t_, senders_w_t_, receivers_v_t_, receivers_w_t_,
           node_latent, senders_pos, receivers_pos, node_type, params):
    senders = edge_index[0]
    receivers = edge_index[1]
    tcol = node_type[:, -1]
    ea0 = edge_attr[:, 0]
    k2w, k4w = _assemble_weights(params)
    vt = jnp.concatenate(
        [vector_a.T, vector_b.T, vector_c.T, senders_v_t_.T, senders_w_t_.T,
         receivers_v_t_.T, receivers_w_t_.T, edge_dx_.T, senders_pos.T,
         receivers_pos.T, edge_attr.T], axis=0)

    nlsum, w = _get_k1()(node_latent, tcol, senders, receivers, ea0)
    w3 = w.reshape(E // BE, 1, BE)
    payload, eoT = _run_k2(vt, w3, nlsum, k2w)
    S2 = _get_k3()(payload, receivers, jnp.zeros((N, 16), f32))
    dv, dw, mfa = _run_k4(S2, node_latent, k4w)
    fij, tauij = _get_k5()(mfa, receivers, eoT)
    return (fij, tauij, dv, dw)


# speedup vs baseline: 6.6725x; 1.0429x over previous
import functools

import jax
import jax.numpy as jnp
from jax import lax
from jax.experimental import pallas as pl
from jax.experimental.pallas import tpu as pltpu
from jax.experimental.pallas import tpu_sc as plsc

N = 10000
E = 320000
D = 128

NC = 2
NS = 16
NW = NC * NS
EC = E // NW
GCH = 400
SCH3 = 2000
CH5 = 400

BE = 3200
BN = 1000

f32 = jnp.float32
i32 = jnp.int32


def _k1_body(nl_hbm, tcol_hbm, s_hbm, r_hbm, ea0_hbm, nlsum_hbm, w_hbm,
             tcol_v, sidx_v, ridx_v, rows_v, ea_v, w_v, sem):
    core = lax.axis_index("c")
    sub = lax.axis_index("s")
    wid = sub * NC + core
    base_w = wid * EC
    pltpu.sync_copy(tcol_hbm, tcol_v)

    def chunk(ci, carry):
        base = pl.multiple_of(base_w + ci * GCH, GCH)
        pltpu.sync_copy(s_hbm.at[pl.ds(base, GCH)], sidx_v)
        pltpu.sync_copy(r_hbm.at[pl.ds(base, GCH)], ridx_v)
        pltpu.sync_copy(ea0_hbm.at[pl.ds(base, GCH)], ea_v)
        pltpu.async_copy(nl_hbm.at[sidx_v], rows_v, sem).wait()
        pltpu.async_copy(nl_hbm.at[ridx_v], rows_v, sem, add=True).wait()
        pltpu.sync_copy(rows_v, nlsum_hbm.at[pl.ds(base, GCH), :])
        one = jnp.ones((16,), f32)
        zero = jnp.zeros((16,), f32)
        for i in range(GCH // 16):
            sl = pl.ds(i * 16, 16)
            gs = plsc.load_gather(tcol_v, [sidx_v[sl]])
            gr = plsc.load_gather(tcol_v, [ridx_v[sl]])
            hit = ((ea_v[sl] == -1.0) & (gr == -1.0)) & (gs != -1.0)
            w_v[sl] = jnp.where(hit, one, zero)
        pltpu.sync_copy(w_v, w_hbm.at[pl.ds(base, GCH)])
        return carry

    lax.fori_loop(0, EC // GCH, chunk, 0)


@functools.cache
def _get_k1():
  return functools.partial(
    pl.kernel,
    out_type=(jax.ShapeDtypeStruct((E, D), f32),
              jax.ShapeDtypeStruct((E,), f32)),
    mesh=plsc.VectorSubcoreMesh(core_axis_name="c", subcore_axis_name="s",
                                num_cores=NC, num_subcores=NS),
    scratch_types=[pltpu.VMEM((N,), f32),
                   pltpu.VMEM((GCH,), i32),
                   pltpu.VMEM((GCH,), i32),
                   pltpu.VMEM((GCH, D), f32),
                   pltpu.VMEM((GCH,), f32),
                   pltpu.VMEM((GCH,), f32),
                   pltpu.SemaphoreType.DMA],
    compiler_params=pltpu.CompilerParams(needs_layout_passes=False),
)(_k1_body)


def _k3_body(pay_hbm, r_hbm, z_hbm, out_hbm, acc_sh, pay_v, idx_v):
    core = lax.axis_index("c")
    sub = lax.axis_index("s")
    wid = sub * NC + core

    @pl.when(sub < 10)
    def _zero():
        pltpu.sync_copy(z_hbm.at[pl.ds(sub * 1000, 1000), :],
                        acc_sh.at[pl.ds(sub * 1000, 1000), :])
    plsc.subcore_barrier()

    def chunk(ci, carry):
        base = pl.multiple_of(wid * EC + ci * SCH3, SCH3)
        pltpu.sync_copy(pay_hbm.at[pl.ds(base, SCH3), :], pay_v)
        pltpu.sync_copy(r_hbm.at[pl.ds(base, SCH3)], idx_v)
        pltpu.sync_copy(pay_v, acc_sh.at[idx_v], add=True)
        return carry

    lax.fori_loop(0, EC // SCH3, chunk, 0)
    plsc.subcore_barrier()

    @pl.when(sub < 10)
    def _writeout():
        pltpu.sync_copy(acc_sh.at[pl.ds(sub * 1000, 1000), :],
                        out_hbm.at[pl.ds(core * N + sub * 1000, 1000), :])


@functools.cache
def _get_k3():
  return functools.partial(
    pl.kernel,
    out_type=jax.ShapeDtypeStruct((2 * N, 16), f32),
    mesh=plsc.VectorSubcoreMesh(core_axis_name="c", subcore_axis_name="s",
                                num_cores=NC, num_subcores=NS),
    scratch_types=[pltpu.VMEM_SHARED((N, 16), f32),
                   pltpu.VMEM((SCH3, 16), f32),
                   pltpu.VMEM((SCH3,), i32)],
    compiler_params=pltpu.CompilerParams(use_tc_tiling_on_sc=False),
)(_k3_body)


def _k5_body(mfa_hbm, r_hbm, eoT_hbm, fij_hbm, tau_hbm,
             mfa_v, idx_v, eo_v, f_v, t_v):
    core = lax.axis_index("c")
    sub = lax.axis_index("s")
    wid = sub * NC + core
    pltpu.sync_copy(mfa_hbm, mfa_v)
    c0 = jnp.zeros((16,), i32)

    def chunk(ci, carry):
        base = pl.multiple_of(wid * EC + ci * CH5, CH5)
        pltpu.sync_copy(r_hbm.at[pl.ds(base, CH5)], idx_v)
        pltpu.sync_copy(eoT_hbm.at[:, pl.ds(base, CH5)], eo_v)
        for k in range(CH5 // 16):
            sl = pl.ds(k * 16, 16)
            rid = lax.iota(i32, 16) + (k * 16)
            ridx = idx_v[sl]
            e0 = eo_v[0, sl]
            e1 = eo_v[1, sl]
            e2 = eo_v[2, sl]
            a0 = eo_v[3, sl]
            a1 = eo_v[4, sl]
            a2 = eo_v[5, sl]
            lam = eo_v[6, sl]
            w = eo_v[7, sl]
            l0 = eo_v[8, sl]
            l1 = eo_v[9, sl]
            l2 = eo_v[10, sl]
            mf0 = plsc.load_gather(mfa_v, [ridx, c0])
            mf1 = plsc.load_gather(mfa_v, [ridx, c0 + 1])
            mf2 = plsc.load_gather(mfa_v, [ridx, c0 + 2])
            ma0 = plsc.load_gather(mfa_v, [ridx, c0 + 3])
            ma1 = plsc.load_gather(mfa_v, [ridx, c0 + 4])
            ma2 = plsc.load_gather(mfa_v, [ridx, c0 + 5])
            f0 = e0 - mf0 * w
            f1 = e1 - mf1 * w
            f2 = e2 - mf2 * w
            g0 = f0 * lam
            g1 = f1 * lam
            g2 = f2 * lam
            t0 = (a0 - ma0 * w) - (l1 * g2 - l2 * g1)
            t1 = (a1 - ma1 * w) - (l2 * g0 - l0 * g2)
            t2 = (a2 - ma2 * w) - (l0 * g1 - l1 * g0)
            plsc.store_scatter(f_v, [rid, c0], f0)
            plsc.store_scatter(f_v, [rid, c0 + 1], f1)
            plsc.store_scatter(f_v, [rid, c0 + 2], f2)
            plsc.store_scatter(t_v, [rid, c0], t0)
            plsc.store_scatter(t_v, [rid, c0 + 1], t1)
            plsc.store_scatter(t_v, [rid, c0 + 2], t2)
        pltpu.sync_copy(f_v, fij_hbm.at[pl.ds(base, CH5), :])
        pltpu.sync_copy(t_v, tau_hbm.at[pl.ds(base, CH5), :])
        return carry

    lax.fori_loop(0, EC // CH5, chunk, 0)


@functools.cache
def _get_k5():
  return functools.partial(
    pl.kernel,
    out_type=(jax.ShapeDtypeStruct((E, 3), f32),
              jax.ShapeDtypeStruct((E, 3), f32)),
    mesh=plsc.VectorSubcoreMesh(core_axis_name="c", subcore_axis_name="s",
                                num_cores=NC, num_subcores=NS),
    scratch_types=[pltpu.VMEM((N, 8), f32),
                   pltpu.VMEM((CH5,), i32),
                   pltpu.VMEM((16, CH5), f32),
                   pltpu.VMEM((CH5, 3), f32),
                   pltpu.VMEM((CH5, 3), f32)],
    compiler_params=pltpu.CompilerParams(use_tc_tiling_on_sc=False,
                                         needs_layout_passes=False),
)(_k5_body)


def _ln(x, g, b):
    mu = jnp.mean(x, axis=-1, keepdims=True)
    var = jnp.mean((x - mu) ** 2, axis=-1, keepdims=True)
    return (x - mu) / jnp.sqrt(var + 1e-5) * g + b


def _cross(u, v):
    return jnp.concatenate([
        u[:, 1:2] * v[:, 2:3] - u[:, 2:3] * v[:, 1:2],
        u[:, 2:3] * v[:, 0:1] - u[:, 0:1] * v[:, 2:3],
        u[:, 0:1] * v[:, 1:2] - u[:, 1:2] * v[:, 0:1]], axis=1)


def _mm(a, b):
    return jnp.dot(a.astype(jnp.bfloat16), b.astype(jnp.bfloat16),
                   preferred_element_type=f32)


def _k2_body(vt, w3, nls,
             neW1, neb1, neW2, neb2, neg, nebe,
             eeW1, eeb1, eeW2, eeb2, eeg, eebe,
             ieW1, ieb1, ieW2, ieb2, ieg, iebe,
             dW1, db1, dW2, db2,
             payload, eoT):
    V = vt[...]

    def row(j):
        return V[j:j + 1, :]

    def tdot(i, j):
        return (row(i) * row(j) + row(i + 1) * row(j + 1)
                + row(i + 2) * row(j + 2))

    sfT = jnp.concatenate([tdot(0, 9), tdot(3, 9), tdot(6, 9),
                           tdot(0, 12), tdot(3, 12), tdot(6, 12)], axis=0)
    rfT = jnp.concatenate([tdot(0, 15), tdot(3, 15), tdot(6, 15),
                           tdot(0, 18), tdot(3, 18), tdot(6, 18)], axis=0)
    both = jnp.concatenate([sfT.T, -rfT.T], axis=0)
    h = jnp.maximum(_mm(both, neW1[...]) + neb1[...], 0.0)
    o = _ln(_mm(h, neW2[...]) + neb2[...], neg[...], nebe[...])
    spr = o[:BE] + o[BE:]

    nrm = jnp.sqrt(tdot(21, 21))
    efT = jnp.concatenate([nrm, V[30:34, :]], axis=0)
    he = jnp.maximum(_mm(efT.T, eeW1[...]) + eeb1[...], 0.0)
    el = _ln(_mm(he, eeW2[...]) + eeb2[...], eeg[...], eebe[...])

    X = jnp.concatenate([spr, nls[...], el], axis=1)
    hi = jnp.maximum(_mm(X, ieW1[...]) + ieb1[...], 0.0)
    il = _ln(_mm(hi, ieW2[...]) + ieb2[...], ieg[...], iebe[...])

    Hd = jnp.maximum(_mm(il, dW1[...]) + db1[...], 0.0)
    C = (_mm(Hd, dW2[...]) + db2[...]).T

    def crow(j):
        return C[j:j + 1, :]

    f = [crow(0) * row(c) + crow(1) * row(3 + c) + crow(2) * row(6 + c)
         for c in range(3)]
    a = [crow(8) * row(c) + crow(9) * row(3 + c) + crow(10) * row(6 + c)
         for c in range(3)]
    lam = crow(16)
    w = w3[...][0]
    lv = [row(24 + c) - row(27 + c) for c in range(3)]
    g = [f[c] * lam for c in range(3)]
    mom = [lv[1] * g[2] - lv[2] * g[1],
           lv[2] * g[0] - lv[0] * g[2],
           lv[0] * g[1] - lv[1] * g[0]]
    omw = 1.0 - w
    wl = w * lam
    payT = jnp.concatenate(
        [w, w * f[0], w * f[1], w * f[2], w * a[0], w * a[1], w * a[2],
         omw * f[0], omw * f[1], omw * f[2],
         omw * a[0] - mom[0], omw * a[1] - mom[1], omw * a[2] - mom[2],
         lv[0] * wl, lv[1] * wl, lv[2] * wl], axis=0)
    payload[...] = payT.T
    eoT[...] = jnp.concatenate(
        [f[0], f[1], f[2], a[0], a[1], a[2], lam, w,
         lv[0], lv[1], lv[2], jnp.zeros((5, BE), f32)], axis=0)


def _run_k2(vt, w3, nls, wts):
    g = E // BE

    def wspec(a):
        return pl.BlockSpec(a.shape, lambda i: tuple(0 for _ in a.shape))

    in_specs = [pl.BlockSpec((34, BE), lambda i: (0, i)),
                pl.BlockSpec((1, 1, BE), lambda i: (i, 0, 0)),
                pl.BlockSpec((BE, D), lambda i: (i, 0))]
    in_specs += [wspec(a) for a in wts]
    return pl.pallas_call(
        _k2_body,
        grid=(g,),
        in_specs=in_specs,
        out_specs=[pl.BlockSpec((BE, 16), lambda i: (i, 0)),
                   pl.BlockSpec((16, BE), lambda i: (0, i))],
        out_shape=[jax.ShapeDtypeStruct((E, 16), f32),
                   jax.ShapeDtypeStruct((16, E), f32)],
    )(vt, w3, nls, *wts)


def _k4_body(Sa, Sb, nl, mW1, mb1, mW2, mb2, dv, dw, mfa):
    S = Sa[...] + Sb[...]
    cnt = S[:, 0:1]
    denom = jnp.maximum(cnt, 1.0)
    mean_f = S[:, 1:4] / denom
    mean_a = S[:, 4:7] / denom
    nf = S[:, 7:10]
    ntq = S[:, 10:13] + _cross(S[:, 13:16], mean_f)
    h = jnp.maximum(_mm(nl[...], mW1[...]) + mb1[...], 0.0)
    Cn = _mm(h, mW2[...]) + mb2[...]
    dv[...] = Cn[:, 0:1] * nf
    dw[...] = Cn[:, 8:9] * ntq
    mfa[...] = jnp.concatenate([mean_f, mean_a, jnp.zeros((BN, 2), f32)],
                               axis=1)


def _run_k4(S2, nl, wts):
    g = N // BN
    sspec = pl.BlockSpec((BN, 16), lambda j: (j, 0))
    sspec2 = pl.BlockSpec((BN, 16), lambda j: (j + N // BN, 0))

    def wspec(a):
        return pl.BlockSpec(a.shape, lambda j: tuple(0 for _ in a.shape))

    return pl.pallas_call(
        _k4_body,
        grid=(g,),
        in_specs=[sspec, sspec2, pl.BlockSpec((BN, D), lambda j: (j, 0))]
                 + [wspec(a) for a in wts],
        out_specs=[pl.BlockSpec((BN, 3), lambda j: (j, 0)),
                   pl.BlockSpec((BN, 3), lambda j: (j, 0)),
                   pl.BlockSpec((BN, 8), lambda j: (j, 0))],
        out_shape=[jax.ShapeDtypeStruct((N, 3), f32),
                   jax.ShapeDtypeStruct((N, 3), f32),
                   jax.ShapeDtypeStruct((N, 8), f32)],
    )(S2, S2, nl, *wts)


def _row(v):
    return v.reshape(1, -1)


def _assemble_weights(params):
    ne = params["node_enc"]
    ee = params["edge_enc"]
    ie = params["inter_enc"]
    i1 = params["i1"]
    i2 = params["i2"]
    fs = params["fs"]
    k2w = [ne["W1"], _row(ne["b1"]), ne["W2"], _row(ne["b2"]),
           _row(ne["g"]), _row(ne["beta"]),
           ee["W1"], _row(ee["b1"]), ee["W2"], _row(ee["b2"]),
           _row(ee["g"]), _row(ee["beta"]),
           ie["W1"], _row(ie["b1"]), ie["W2"], _row(ie["b2"]),
           _row(ie["g"]), _row(ie["beta"])]
    dW1 = jnp.concatenate([i1["W1"], i2["W1"], fs["W1"]], axis=1)
    db1 = _row(jnp.concatenate([i1["b1"], i2["b1"], fs["b1"]]))
    dW2 = jnp.zeros((3 * D, 24), f32)
    dW2 = dW2.at[0:D, 0:3].set(i1["W2"])
    dW2 = dW2.at[D:2 * D, 8:11].set(i2["W2"])
    dW2 = dW2.at[2 * D:3 * D, 16:17].set(fs["W2"])
    db2 = jnp.zeros((24,), f32)
    db2 = db2.at[0:3].set(i1["b2"])
    db2 = db2.at[8:11].set(i2["b2"])
    db2 = db2.at[16:17].set(fs["b2"])
    k2w += [dW1, db1, dW2, _row(db2)]

    im = params["inv_mass"]
    ii = params["inv_inertia"]
    mW1 = jnp.concatenate([im["W1"], ii["W1"]], axis=1)
    mb1 = _row(jnp.concatenate([im["b1"], ii["b1"]]))
    mW2 = jnp.zeros((2 * D, 16), f32)
    mW2 = mW2.at[0:D, 0:1].set(im["W2"])
    mW2 = mW2.at[D:2 * D, 8:9].set(ii["W2"])
    mb2 = jnp.zeros((16,), f32)
    mb2 = mb2.at[0:1].set(im["b2"])
    mb2 = mb2.at[8:9].set(ii["b2"])
    k4w = [mW1, mb1, mW2, _row(mb2)]
    return k2w, k4w


def kernel(edge_index, edge_dx_, edge_attr, vector_a, vector_b, vector_c,
           senders_v_t_, senders_w_t_, receivers_v_t_, receivers_w_t_,
           node_latent, senders_pos, receivers_pos, node_type, params):
    senders = edge_index[0]
    receivers = edge_index[1]
    tcol = node_type[:, -1]
    ea0 = edge_attr[:, 0]
    k2w, k4w = _assemble_weights(params)
    vt = jnp.concatenate(
        [vector_a.T, vector_b.T, vector_c.T, senders_v_t_.T, senders_w_t_.T,
         receivers_v_t_.T, receivers_w_t_.T, edge_dx_.T, senders_pos.T,
         receivers_pos.T, edge_attr.T], axis=0)

    nlsum, w = _get_k1()(node_latent, tcol, senders, receivers, ea0)
    w3 = w.reshape(E // BE, 1, BE)
    payload, eoT = _run_k2(vt, w3, nlsum, k2w)
    S2 = _get_k3()(payload, receivers, jnp.zeros((N, 16), f32))
    dv, dw, mfa = _run_k4(S2, node_latent, k4w)
    fij, tauij = _get_k5()(mfa, receivers, eoT)
    return (fij, tauij, dv, dw)


# speedup vs baseline: 7.0052x; 1.0499x over previous
import functools

import jax
import jax.numpy as jnp
from jax import lax
from jax.experimental import pallas as pl
from jax.experimental.pallas import tpu as pltpu
from jax.experimental.pallas import tpu_sc as plsc

N = 10000
E = 320000
D = 128

NC = 2
NS = 16
NW = NC * NS
EC = E // NW
GCH = 400
SCH3 = 2000
CH5 = 400

BE = 3200
BN = 1000

f32 = jnp.float32
i32 = jnp.int32


def _k1_body(nl_hbm, tcol_hbm, s_hbm, r_hbm, ea0_hbm, nlsum_hbm, w_hbm,
             tcol_v, sidx_v, ridx_v, rows_v, ea_v, w_v, sem):
    core = lax.axis_index("c")
    sub = lax.axis_index("s")
    wid = sub * NC + core
    base_w = wid * EC
    pltpu.sync_copy(tcol_hbm, tcol_v)

    def chunk(ci, carry):
        base = pl.multiple_of(base_w + ci * GCH, GCH)
        pltpu.sync_copy(s_hbm.at[pl.ds(base, GCH)], sidx_v)
        pltpu.sync_copy(r_hbm.at[pl.ds(base, GCH)], ridx_v)
        pltpu.sync_copy(ea0_hbm.at[pl.ds(base, GCH)], ea_v)
        pltpu.async_copy(nl_hbm.at[sidx_v], rows_v, sem).wait()
        pltpu.async_copy(nl_hbm.at[ridx_v], rows_v, sem, add=True).wait()
        pltpu.sync_copy(rows_v, nlsum_hbm.at[pl.ds(base, GCH), :])
        one = jnp.ones((16,), f32)
        zero = jnp.zeros((16,), f32)
        for i in range(GCH // 16):
            sl = pl.ds(i * 16, 16)
            gs = plsc.load_gather(tcol_v, [sidx_v[sl]])
            gr = plsc.load_gather(tcol_v, [ridx_v[sl]])
            hit = ((ea_v[sl] == -1.0) & (gr == -1.0)) & (gs != -1.0)
            w_v[sl] = jnp.where(hit, one, zero)
        pltpu.sync_copy(w_v, w_hbm.at[pl.ds(base, GCH)])
        return carry

    lax.fori_loop(0, EC // GCH, chunk, 0)


@functools.cache
def _get_k1():
  return functools.partial(
    pl.kernel,
    out_type=(jax.ShapeDtypeStruct((E, D), f32),
              jax.ShapeDtypeStruct((E,), f32)),
    mesh=plsc.VectorSubcoreMesh(core_axis_name="c", subcore_axis_name="s",
                                num_cores=NC, num_subcores=NS),
    scratch_types=[pltpu.VMEM((N,), f32),
                   pltpu.VMEM((GCH,), i32),
                   pltpu.VMEM((GCH,), i32),
                   pltpu.VMEM((GCH, D), f32),
                   pltpu.VMEM((GCH,), f32),
                   pltpu.VMEM((GCH,), f32),
                   pltpu.SemaphoreType.DMA],
    compiler_params=pltpu.CompilerParams(needs_layout_passes=False),
)(_k1_body)


def _k3_body(pay_hbm, r_hbm, z_hbm, out_hbm, acc_sh, pay_v, idx_v):
    core = lax.axis_index("c")
    sub = lax.axis_index("s")
    wid = sub * NC + core

    @pl.when(sub < 10)
    def _zero():
        pltpu.sync_copy(z_hbm.at[pl.ds(sub * 1000, 1000), :],
                        acc_sh.at[pl.ds(sub * 1000, 1000), :])
    plsc.subcore_barrier()

    def chunk(ci, carry):
        base = pl.multiple_of(wid * EC + ci * SCH3, SCH3)
        pltpu.sync_copy(pay_hbm.at[pl.ds(base, SCH3), :], pay_v)
        pltpu.sync_copy(r_hbm.at[pl.ds(base, SCH3)], idx_v)
        pltpu.sync_copy(pay_v, acc_sh.at[idx_v], add=True)
        return carry

    lax.fori_loop(0, EC // SCH3, chunk, 0)
    plsc.subcore_barrier()

    @pl.when(sub < 10)
    def _writeout():
        pltpu.sync_copy(acc_sh.at[pl.ds(sub * 1000, 1000), :],
                        out_hbm.at[pl.ds(core * N + sub * 1000, 1000), :])


@functools.cache
def _get_k3():
  return functools.partial(
    pl.kernel,
    out_type=jax.ShapeDtypeStruct((2 * N, 16), f32),
    mesh=plsc.VectorSubcoreMesh(core_axis_name="c", subcore_axis_name="s",
                                num_cores=NC, num_subcores=NS),
    scratch_types=[pltpu.VMEM_SHARED((N, 16), f32),
                   pltpu.VMEM((SCH3, 16), f32),
                   pltpu.VMEM((SCH3,), i32)],
    compiler_params=pltpu.CompilerParams(use_tc_tiling_on_sc=False),
)(_k3_body)


def _k5_body(mfa_hbm, r_hbm, eoT_hbm, fij_hbm, tau_hbm,
             mfa_v, idx_v, eo_v, f_v, t_v):
    core = lax.axis_index("c")
    sub = lax.axis_index("s")
    wid = sub * NC + core
    pltpu.sync_copy(mfa_hbm, mfa_v)
    c0 = jnp.zeros((16,), i32)

    def chunk(ci, carry):
        base = pl.multiple_of(wid * EC + ci * CH5, CH5)
        pltpu.sync_copy(r_hbm.at[pl.ds(base, CH5)], idx_v)
        pltpu.sync_copy(eoT_hbm.at[:, pl.ds(base, CH5)], eo_v)
        for k in range(CH5 // 16):
            sl = pl.ds(k * 16, 16)
            rid = lax.iota(i32, 16) + (k * 16)
            ridx = idx_v[sl]
            e0 = eo_v[0, sl]
            e1 = eo_v[1, sl]
            e2 = eo_v[2, sl]
            a0 = eo_v[3, sl]
            a1 = eo_v[4, sl]
            a2 = eo_v[5, sl]
            lam = eo_v[6, sl]
            w = eo_v[7, sl]
            l0 = eo_v[8, sl]
            l1 = eo_v[9, sl]
            l2 = eo_v[10, sl]
            mf0 = plsc.load_gather(mfa_v, [ridx, c0])
            mf1 = plsc.load_gather(mfa_v, [ridx, c0 + 1])
            mf2 = plsc.load_gather(mfa_v, [ridx, c0 + 2])
            ma0 = plsc.load_gather(mfa_v, [ridx, c0 + 3])
            ma1 = plsc.load_gather(mfa_v, [ridx, c0 + 4])
            ma2 = plsc.load_gather(mfa_v, [ridx, c0 + 5])
            f0 = e0 - mf0 * w
            f1 = e1 - mf1 * w
            f2 = e2 - mf2 * w
            g0 = f0 * lam
            g1 = f1 * lam
            g2 = f2 * lam
            t0 = (a0 - ma0 * w) - (l1 * g2 - l2 * g1)
            t1 = (a1 - ma1 * w) - (l2 * g0 - l0 * g2)
            t2 = (a2 - ma2 * w) - (l0 * g1 - l1 * g0)
            plsc.store_scatter(f_v, [rid, c0], f0)
            plsc.store_scatter(f_v, [rid, c0 + 1], f1)
            plsc.store_scatter(f_v, [rid, c0 + 2], f2)
            plsc.store_scatter(t_v, [rid, c0], t0)
            plsc.store_scatter(t_v, [rid, c0 + 1], t1)
            plsc.store_scatter(t_v, [rid, c0 + 2], t2)
        pltpu.sync_copy(f_v, fij_hbm.at[pl.ds(base, CH5), :])
        pltpu.sync_copy(t_v, tau_hbm.at[pl.ds(base, CH5), :])
        return carry

    lax.fori_loop(0, EC // CH5, chunk, 0)


@functools.cache
def _get_k5():
  return functools.partial(
    pl.kernel,
    out_type=(jax.ShapeDtypeStruct((E, 3), f32),
              jax.ShapeDtypeStruct((E, 3), f32)),
    mesh=plsc.VectorSubcoreMesh(core_axis_name="c", subcore_axis_name="s",
                                num_cores=NC, num_subcores=NS),
    scratch_types=[pltpu.VMEM((N, 8), f32),
                   pltpu.VMEM((CH5,), i32),
                   pltpu.VMEM((16, CH5), f32),
                   pltpu.VMEM((CH5, 3), f32),
                   pltpu.VMEM((CH5, 3), f32)],
    compiler_params=pltpu.CompilerParams(use_tc_tiling_on_sc=False,
                                         needs_layout_passes=False),
)(_k5_body)


def _ln(x, g, b):
    mu = jnp.mean(x, axis=-1, keepdims=True)
    var = jnp.mean((x - mu) ** 2, axis=-1, keepdims=True)
    return (x - mu) / jnp.sqrt(var + 1e-5) * g + b


def _cross(u, v):
    return jnp.concatenate([
        u[:, 1:2] * v[:, 2:3] - u[:, 2:3] * v[:, 1:2],
        u[:, 2:3] * v[:, 0:1] - u[:, 0:1] * v[:, 2:3],
        u[:, 0:1] * v[:, 1:2] - u[:, 1:2] * v[:, 0:1]], axis=1)


def _mm(a, b):
    return jnp.dot(a.astype(jnp.bfloat16), b.astype(jnp.bfloat16),
                   preferred_element_type=f32)


def _dgT(aT, b):
    return lax.dot_general(aT.astype(jnp.bfloat16), b.astype(jnp.bfloat16),
                           (((0,), (0,)), ((), ())),
                           preferred_element_type=f32)


def _dgoT(b, a):
    return lax.dot_general(b.astype(jnp.bfloat16), a.astype(jnp.bfloat16),
                           (((0,), (1,)), ((), ())),
                           preferred_element_type=f32)


def _k2_body(vt, w3, nls,
             neW1, neb1, neW2, neb2, neg, nebe,
             eeW1, eeb1, eeW2, eeb2, eeg, eebe,
             ieW1, ieb1, ieW2, ieb2, ieg, iebe,
             dW1, db1, dW2, db2,
             payload, eoT):
    V = vt[...]

    def row(j):
        return V[j:j + 1, :]

    def tdot(i, j):
        return (row(i) * row(j) + row(i + 1) * row(j + 1)
                + row(i + 2) * row(j + 2))

    sfT = jnp.concatenate([tdot(0, 9), tdot(3, 9), tdot(6, 9),
                           tdot(0, 12), tdot(3, 12), tdot(6, 12)], axis=0)
    nrfT = -jnp.concatenate([tdot(0, 15), tdot(3, 15), tdot(6, 15),
                             tdot(0, 18), tdot(3, 18), tdot(6, 18)], axis=0)
    hs = jnp.maximum(_dgT(sfT, neW1[...]) + neb1[...], 0.0)
    os_ = _ln(_mm(hs, neW2[...]) + neb2[...], neg[...], nebe[...])
    hr = jnp.maximum(_dgT(nrfT, neW1[...]) + neb1[...], 0.0)
    or_ = _ln(_mm(hr, neW2[...]) + neb2[...], neg[...], nebe[...])
    spr = os_ + or_

    nrm = jnp.sqrt(tdot(21, 21))
    efT = jnp.concatenate([nrm, V[30:34, :]], axis=0)
    he = jnp.maximum(_dgT(efT, eeW1[...]) + eeb1[...], 0.0)
    el = _ln(_mm(he, eeW2[...]) + eeb2[...], eeg[...], eebe[...])

    X = jnp.concatenate([spr, nls[...], el], axis=1)
    hi = jnp.maximum(_mm(X, ieW1[...]) + ieb1[...], 0.0)
    il = _ln(_mm(hi, ieW2[...]) + ieb2[...], ieg[...], iebe[...])

    Hd = jnp.maximum(_mm(il, dW1[...]) + db1[...], 0.0)
    C = _dgoT(dW2[...], Hd) + db2[...]

    def crow(j):
        return C[j:j + 1, :]

    f = [crow(0) * row(c) + crow(1) * row(3 + c) + crow(2) * row(6 + c)
         for c in range(3)]
    a = [crow(8) * row(c) + crow(9) * row(3 + c) + crow(10) * row(6 + c)
         for c in range(3)]
    lam = crow(16)
    w = w3[...][0]
    lv = [row(24 + c) - row(27 + c) for c in range(3)]
    g = [f[c] * lam for c in range(3)]
    mom = [lv[1] * g[2] - lv[2] * g[1],
           lv[2] * g[0] - lv[0] * g[2],
           lv[0] * g[1] - lv[1] * g[0]]
    omw = 1.0 - w
    wl = w * lam
    payT = jnp.concatenate(
        [w, w * f[0], w * f[1], w * f[2], w * a[0], w * a[1], w * a[2],
         omw * f[0], omw * f[1], omw * f[2],
         omw * a[0] - mom[0], omw * a[1] - mom[1], omw * a[2] - mom[2],
         lv[0] * wl, lv[1] * wl, lv[2] * wl], axis=0)
    payload[...] = payT.T
    eoT[...] = jnp.concatenate(
        [f[0], f[1], f[2], a[0], a[1], a[2], lam, w,
         lv[0], lv[1], lv[2], jnp.zeros((5, BE), f32)], axis=0)


def _run_k2(vt, w3, nls, wts):
    g = E // BE

    def wspec(a):
        return pl.BlockSpec(a.shape, lambda i: tuple(0 for _ in a.shape))

    in_specs = [pl.BlockSpec((34, BE), lambda i: (0, i)),
                pl.BlockSpec((1, 1, BE), lambda i: (i, 0, 0)),
                pl.BlockSpec((BE, D), lambda i: (i, 0))]
    in_specs += [wspec(a) for a in wts]
    return pl.pallas_call(
        _k2_body,
        grid=(g,),
        in_specs=in_specs,
        out_specs=[pl.BlockSpec((BE, 16), lambda i: (i, 0)),
                   pl.BlockSpec((16, BE), lambda i: (0, i))],
        out_shape=[jax.ShapeDtypeStruct((E, 16), f32),
                   jax.ShapeDtypeStruct((16, E), f32)],
    )(vt, w3, nls, *wts)


def _k4_body(Sa, Sb, nl, mW1, mb1, mW2, mb2, dv, dw, mfa):
    S = Sa[...] + Sb[...]
    cnt = S[:, 0:1]
    denom = jnp.maximum(cnt, 1.0)
    mean_f = S[:, 1:4] / denom
    mean_a = S[:, 4:7] / denom
    nf = S[:, 7:10]
    ntq = S[:, 10:13] + _cross(S[:, 13:16], mean_f)
    h = jnp.maximum(_mm(nl[...], mW1[...]) + mb1[...], 0.0)
    Cn = _mm(h, mW2[...]) + mb2[...]
    dv[...] = Cn[:, 0:1] * nf
    dw[...] = Cn[:, 8:9] * ntq
    mfa[...] = jnp.concatenate([mean_f, mean_a, jnp.zeros((BN, 2), f32)],
                               axis=1)


def _run_k4(S2, nl, wts):
    g = N // BN
    sspec = pl.BlockSpec((BN, 16), lambda j: (j, 0))
    sspec2 = pl.BlockSpec((BN, 16), lambda j: (j + N // BN, 0))

    def wspec(a):
        return pl.BlockSpec(a.shape, lambda j: tuple(0 for _ in a.shape))

    return pl.pallas_call(
        _k4_body,
        grid=(g,),
        in_specs=[sspec, sspec2, pl.BlockSpec((BN, D), lambda j: (j, 0))]
                 + [wspec(a) for a in wts],
        out_specs=[pl.BlockSpec((BN, 3), lambda j: (j, 0)),
                   pl.BlockSpec((BN, 3), lambda j: (j, 0)),
                   pl.BlockSpec((BN, 8), lambda j: (j, 0))],
        out_shape=[jax.ShapeDtypeStruct((N, 3), f32),
                   jax.ShapeDtypeStruct((N, 3), f32),
                   jax.ShapeDtypeStruct((N, 8), f32)],
    )(S2, S2, nl, *wts)


def _row(v):
    return v.reshape(1, -1)


def _assemble_weights(params):
    ne = params["node_enc"]
    ee = params["edge_enc"]
    ie = params["inter_enc"]
    i1 = params["i1"]
    i2 = params["i2"]
    fs = params["fs"]
    k2w = [ne["W1"], _row(ne["b1"]), ne["W2"], _row(ne["b2"]),
           _row(ne["g"]), _row(ne["beta"]),
           ee["W1"], _row(ee["b1"]), ee["W2"], _row(ee["b2"]),
           _row(ee["g"]), _row(ee["beta"]),
           ie["W1"], _row(ie["b1"]), ie["W2"], _row(ie["b2"]),
           _row(ie["g"]), _row(ie["beta"])]
    dW1 = jnp.concatenate([i1["W1"], i2["W1"], fs["W1"]], axis=1)
    db1 = _row(jnp.concatenate([i1["b1"], i2["b1"], fs["b1"]]))
    dW2 = jnp.zeros((3 * D, 24), f32)
    dW2 = dW2.at[0:D, 0:3].set(i1["W2"])
    dW2 = dW2.at[D:2 * D, 8:11].set(i2["W2"])
    dW2 = dW2.at[2 * D:3 * D, 16:17].set(fs["W2"])
    db2 = jnp.zeros((24,), f32)
    db2 = db2.at[0:3].set(i1["b2"])
    db2 = db2.at[8:11].set(i2["b2"])
    db2 = db2.at[16:17].set(fs["b2"])
    k2w += [dW1, db1, dW2, db2.reshape(24, 1)]

    im = params["inv_mass"]
    ii = params["inv_inertia"]
    mW1 = jnp.concatenate([im["W1"], ii["W1"]], axis=1)
    mb1 = _row(jnp.concatenate([im["b1"], ii["b1"]]))
    mW2 = jnp.zeros((2 * D, 16), f32)
    mW2 = mW2.at[0:D, 0:1].set(im["W2"])
    mW2 = mW2.at[D:2 * D, 8:9].set(ii["W2"])
    mb2 = jnp.zeros((16,), f32)
    mb2 = mb2.at[0:1].set(im["b2"])
    mb2 = mb2.at[8:9].set(ii["b2"])
    k4w = [mW1, mb1, mW2, _row(mb2)]
    return k2w, k4w


def kernel(edge_index, edge_dx_, edge_attr, vector_a, vector_b, vector_c,
           senders_v_t_, senders_w_t_, receivers_v_t_, receivers_w_t_,
           node_latent, senders_pos, receivers_pos, node_type, params):
    senders = edge_index[0]
    receivers = edge_index[1]
    tcol = node_type[:, -1]
    ea0 = edge_attr[:, 0]
    k2w, k4w = _assemble_weights(params)
    vt = jnp.concatenate(
        [vector_a.T, vector_b.T, vector_c.T, senders_v_t_.T, senders_w_t_.T,
         receivers_v_t_.T, receivers_w_t_.T, edge_dx_.T, senders_pos.T,
         receivers_pos.T, edge_attr.T], axis=0)

    nlsum, w = _get_k1()(node_latent, tcol, senders, receivers, ea0)
    w3 = w.reshape(E // BE, 1, BE)
    payload, eoT = _run_k2(vt, w3, nlsum, k2w)
    S2 = _get_k3()(payload, receivers, jnp.zeros((N, 16), f32))
    dv, dw, mfa = _run_k4(S2, node_latent, k4w)
    fij, tauij = _get_k5()(mfa, receivers, eoT)
    return (fij, tauij, dv, dw)
